# baseline (speedup 1.0000x reference)
import functools

import jax
import jax.numpy as jnp
from jax import lax
from jax.experimental import pallas as pl
from jax.experimental.pallas import tpu as pltpu
from jax.experimental.pallas import tpu_sc as plsc

LIST_LEN = 16384
EMBED_DIM = 64
L = 16


def _sc_partials_kernel(nc, ns, chunk):
    nw = nc * ns
    mesh = plsc.VectorSubcoreMesh(core_axis_name="c", subcore_axis_name="s")

    @functools.partial(
        pl.kernel,
        out_type=jax.ShapeDtypeStruct((nw, EMBED_DIM), jnp.float32),
        mesh=mesh,
        scratch_types=[
            pltpu.VMEM((chunk,), jnp.int32),
            pltpu.VMEM((chunk, EMBED_DIM), jnp.float32),
            pltpu.VMEM((EMBED_DIM,), jnp.float32),
            pltpu.SemaphoreType.DMA,
        ],
        compiler_params=pltpu.CompilerParams(use_tc_tiling_on_sc=False),
    )
    def body(item_list, entity_list, word_list,
             item_table, entity_table, word_table,
             out_hbm, idx_v, rows_v, acc_v, sem):
        wid = lax.axis_index("s") * nc + lax.axis_index("c")
        base = wid * chunk

        accs = tuple(jnp.zeros((L,), jnp.float32) for _ in range(EMBED_DIM // L))

        def row_body(i, accs):
            return tuple(
                a + rows_v[i, pl.ds(c * L, L)] for c, a in enumerate(accs)
            )

        for lst, tbl in ((item_list, item_table),
                         (entity_list, entity_table),
                         (word_list, word_table)):
            pltpu.sync_copy(lst.at[pl.ds(base, chunk)], idx_v)
            pltpu.async_copy(tbl.at[idx_v], rows_v, sem).wait()
            accs = lax.fori_loop(0, chunk, row_body, accs)

        for c, a in enumerate(accs):
            acc_v[pl.ds(c * L, L)] = a
        pltpu.sync_copy(acc_v, out_hbm.at[wid])

    return body


def _tc_head_kernel(partials_ref, w_ref, b_ref, out_ref):
    s = jnp.sum(partials_ref[...], axis=0, keepdims=True)
    emb = s * (1.0 / (3.0 * LIST_LEN))
    out_ref[...] = jnp.dot(emb, w_ref[...],
                           preferred_element_type=jnp.float32) + b_ref[...]


def kernel(item_list, entity_list, word_list,
           item_table, entity_table, word_table, W_cls, b_cls):
    info = plsc.get_sparse_core_info()
    nc, ns = info.num_cores, info.num_subcores
    nw = nc * ns
    chunk = LIST_LEN // nw

    partials = _sc_partials_kernel(nc, ns, chunk)(
        item_list.astype(jnp.int32),
        entity_list.astype(jnp.int32),
        word_list.astype(jnp.int32),
        item_table, entity_table, word_table,
    )

    out = pl.pallas_call(
        _tc_head_kernel,
        out_shape=jax.ShapeDtypeStruct((1, 2), jnp.float32),
    )(partials, W_cls, b_cls.reshape(1, 2))
    return out


# speedup vs baseline: 1.1059x; 1.1059x over previous
import functools

import jax
import jax.numpy as jnp
from jax import lax
from jax.experimental import pallas as pl
from jax.experimental.pallas import tpu as pltpu
from jax.experimental.pallas import tpu_sc as plsc

LIST_LEN = 16384
EMBED_DIM = 64
N_ITEM = 1000000
N_ENTITY = 100000
N_WORD = 100000
L = 16

PAD_ITEM = 1000448
PAD_SMALL = 100352
HALF_ITEM = PAD_ITEM // 2
HALF_SMALL = PAD_SMALL // 2
BLK_ITEM = 1024
BLK_SMALL = 2048


def _sc_hist_kernel(nc, ns, chunk):
    mesh = plsc.VectorSubcoreMesh(core_axis_name="c", subcore_axis_name="s")
    sl_item = HALF_ITEM // ns
    sl_small = HALF_SMALL // ns

    @functools.partial(
        pl.kernel,
        out_type=(
            jax.ShapeDtypeStruct((PAD_ITEM,), jnp.float32),
            jax.ShapeDtypeStruct((PAD_SMALL,), jnp.float32),
            jax.ShapeDtypeStruct((PAD_SMALL,), jnp.float32),
        ),
        mesh=mesh,
        scratch_types=[
            pltpu.VMEM((chunk,), jnp.int32),
            pltpu.VMEM((chunk,), jnp.int32),
            pltpu.VMEM((chunk,), jnp.float32),
            pltpu.VMEM((sl_item,), jnp.float32),
            pltpu.VMEM_SHARED((HALF_ITEM + 8,), jnp.float32),
            pltpu.VMEM_SHARED((HALF_SMALL + 8,), jnp.float32),
        ],
        compiler_params=pltpu.CompilerParams(use_tc_tiling_on_sc=False),
    )
    def body(item_list, entity_list, word_list,
             out_item, out_entity, out_word,
             idx_v, loc_v, ones_v, zero_v, h_item, h_small):
        cid = lax.axis_index("c")
        sid = lax.axis_index("s")
        base = sid * chunk

        def fill(i, _):
            zero_v[pl.ds(i * L, L)] = jnp.zeros((L,), jnp.float32)
            return 0

        lax.fori_loop(0, sl_item // L, fill, 0)

        def fill_ones(i, _):
            ones_v[pl.ds(i * L, L)] = jnp.full((L,), 1.0, jnp.float32)
            return 0

        lax.fori_loop(0, chunk // L, fill_ones, 0)

        def rebase(half):
            lo = cid * half

            def step(i, _):
                v = idx_v[pl.ds(i * L, L)] - lo
                oob = (v < 0) | (v >= half)
                loc_v[pl.ds(i * L, L)] = jnp.where(oob, half, v)
                return 0

            lax.fori_loop(0, chunk // L, step, 0)

        pltpu.sync_copy(zero_v, h_item.at[pl.ds(sid * sl_item, sl_item)])
        pltpu.sync_copy(zero_v.at[pl.ds(0, sl_small)],
                        h_small.at[pl.ds(sid * sl_small, sl_small)])

        @pl.when(sid == 0)
        def _():
            pltpu.sync_copy(zero_v.at[pl.ds(0, 8)],
                            h_item.at[pl.ds(HALF_ITEM, 8)])
            pltpu.sync_copy(zero_v.at[pl.ds(0, 8)],
                            h_small.at[pl.ds(HALF_SMALL, 8)])

        plsc.subcore_barrier()

        pltpu.sync_copy(item_list.at[pl.ds(base, chunk)], idx_v)
        rebase(HALF_ITEM)
        pltpu.sync_copy(ones_v, h_item.at[loc_v], add=True)
        pltpu.sync_copy(entity_list.at[pl.ds(base, chunk)], idx_v)
        rebase(HALF_SMALL)
        pltpu.sync_copy(ones_v, h_small.at[loc_v], add=True)
        plsc.subcore_barrier()

        pltpu.sync_copy(h_item.at[pl.ds(sid * sl_item, sl_item)],
                        out_item.at[pl.ds(cid * HALF_ITEM + sid * sl_item,
                                          sl_item)])
        pltpu.sync_copy(h_small.at[pl.ds(sid * sl_small, sl_small)],
                        out_entity.at[pl.ds(cid * HALF_SMALL + sid * sl_small,
                                            sl_small)])
        pltpu.sync_copy(zero_v.at[pl.ds(0, sl_small)],
                        h_small.at[pl.ds(sid * sl_small, sl_small)])
        plsc.subcore_barrier()

        pltpu.sync_copy(word_list.at[pl.ds(base, chunk)], idx_v)
        rebase(HALF_SMALL)
        pltpu.sync_copy(ones_v, h_small.at[loc_v], add=True)
        plsc.subcore_barrier()

        pltpu.sync_copy(h_small.at[pl.ds(sid * sl_small, sl_small)],
                        out_word.at[pl.ds(cid * HALF_SMALL + sid * sl_small,
                                          sl_small)])

    return body


def _mv_item_kernel(tT_ref, c_ref, out_ref):
    j = pl.program_id(0)

    @pl.when(j == 0)
    def _():
        out_ref[...] = jnp.zeros_like(out_ref)

    col = j * BLK_ITEM + lax.broadcasted_iota(jnp.int32, (1, BLK_ITEM), 1)
    prod = jnp.where(col < N_ITEM, tT_ref[...] * c_ref[...], 0.0)
    out_ref[...] += jnp.sum(prod, axis=1, keepdims=True)


def _mv_small_kernel(tTe_ref, tTw_ref, ce_ref, cw_ref, out_ref):
    j = pl.program_id(0)

    @pl.when(j == 0)
    def _():
        out_ref[...] = jnp.zeros_like(out_ref)

    col = j * BLK_SMALL + lax.broadcasted_iota(jnp.int32, (1, BLK_SMALL), 1)
    pe = jnp.where(col < N_ENTITY, tTe_ref[...] * ce_ref[...], 0.0)
    pw = jnp.where(col < N_WORD, tTw_ref[...] * cw_ref[...], 0.0)
    out_ref[...] += jnp.sum(pe + pw, axis=1, keepdims=True)


def _head_kernel(ei_ref, eew_ref, w_ref, b_ref, out_ref):
    s = (ei_ref[...] + eew_ref[...]) * (1.0 / (3.0 * LIST_LEN))
    out_ref[...] = jnp.sum(s * w_ref[...], axis=0, keepdims=True) + b_ref[...]


def kernel(item_list, entity_list, word_list,
           item_table, entity_table, word_table, W_cls, b_cls):
    info = plsc.get_sparse_core_info()
    nc, ns = info.num_cores, info.num_subcores
    chunk = LIST_LEN // ns

    cnt_item, cnt_entity, cnt_word = _sc_hist_kernel(nc, ns, chunk)(
        item_list.astype(jnp.int32),
        entity_list.astype(jnp.int32),
        word_list.astype(jnp.int32),
    )

    e_item = pl.pallas_call(
        _mv_item_kernel,
        grid=(PAD_ITEM // BLK_ITEM,),
        in_specs=[
            pl.BlockSpec((EMBED_DIM, BLK_ITEM), lambda j: (0, j)),
            pl.BlockSpec((BLK_ITEM,), lambda j: (j,)),
        ],
        out_specs=pl.BlockSpec((EMBED_DIM, 1), lambda j: (0, 0)),
        out_shape=jax.ShapeDtypeStruct((EMBED_DIM, 1), jnp.float32),
        compiler_params=pltpu.CompilerParams(
            dimension_semantics=("arbitrary",)),
    )(item_table.T, cnt_item)

    e_ew = pl.pallas_call(
        _mv_small_kernel,
        grid=(PAD_SMALL // BLK_SMALL,),
        in_specs=[
            pl.BlockSpec((EMBED_DIM, BLK_SMALL), lambda j: (0, j)),
            pl.BlockSpec((EMBED_DIM, BLK_SMALL), lambda j: (0, j)),
            pl.BlockSpec((BLK_SMALL,), lambda j: (j,)),
            pl.BlockSpec((BLK_SMALL,), lambda j: (j,)),
        ],
        out_specs=pl.BlockSpec((EMBED_DIM, 1), lambda j: (0, 0)),
        out_shape=jax.ShapeDtypeStruct((EMBED_DIM, 1), jnp.float32),
        compiler_params=pltpu.CompilerParams(
            dimension_semantics=("arbitrary",)),
    )(entity_table.T, word_table.T, cnt_entity, cnt_word)

    out = pl.pallas_call(
        _head_kernel,
        out_shape=jax.ShapeDtypeStruct((1, 2), jnp.float32),
    )(e_item, e_ew, W_cls, b_cls.reshape(1, 2))
    return out


# speedup vs baseline: 4.4304x; 4.0062x over previous
import functools

import jax
import jax.numpy as jnp
from jax import lax
from jax.experimental import pallas as pl
from jax.experimental.pallas import tpu as pltpu
from jax.experimental.pallas import tpu_sc as plsc

LIST_LEN = 16384
EMBED_DIM = 64
N_ITEM = 1000000
N_ENTITY = 100000
N_WORD = 100000
L = 16

BLK_ITEM = 32768
BLK_SMALL = 16384
PAD_ITEM = 31 * BLK_ITEM
PAD_SMALL = 7 * BLK_SMALL
HALF_ITEM = PAD_ITEM // 2
HALF_SMALL = PAD_SMALL // 2


def _sc_hist_kernel(nc, ns, chunk):
    mesh = plsc.VectorSubcoreMesh(core_axis_name="c", subcore_axis_name="s")
    sl_item = HALF_ITEM // ns
    sl_small = HALF_SMALL // ns

    @functools.partial(
        pl.kernel,
        out_type=(
            jax.ShapeDtypeStruct((PAD_ITEM,), jnp.float32),
            jax.ShapeDtypeStruct((PAD_SMALL,), jnp.float32),
            jax.ShapeDtypeStruct((PAD_SMALL,), jnp.float32),
        ),
        mesh=mesh,
        scratch_types=[
            pltpu.VMEM((chunk,), jnp.int32),
            pltpu.VMEM((chunk,), jnp.int32),
            pltpu.VMEM((chunk,), jnp.float32),
            pltpu.VMEM((sl_item,), jnp.float32),
            pltpu.VMEM_SHARED((HALF_ITEM + 8,), jnp.float32),
            pltpu.VMEM_SHARED((HALF_SMALL + 8,), jnp.float32),
        ],
        compiler_params=pltpu.CompilerParams(use_tc_tiling_on_sc=False),
    )
    def body(item_list, entity_list, word_list,
             out_item, out_entity, out_word,
             idx_v, loc_v, ones_v, zero_v, h_item, h_small):
        cid = lax.axis_index("c")
        sid = lax.axis_index("s")
        base = sid * chunk

        def fill(i, _):
            zero_v[pl.ds(i * L, L)] = jnp.zeros((L,), jnp.float32)
            return 0

        lax.fori_loop(0, sl_item // L, fill, 0)

        def fill_ones(i, _):
            ones_v[pl.ds(i * L, L)] = jnp.full((L,), 1.0, jnp.float32)
            return 0

        lax.fori_loop(0, chunk // L, fill_ones, 0)

        def rebase(half):
            lo = cid * half

            def step(i, _):
                v = idx_v[pl.ds(i * L, L)] - lo
                oob = (v < 0) | (v >= half)
                loc_v[pl.ds(i * L, L)] = jnp.where(oob, half, v)
                return 0

            lax.fori_loop(0, chunk // L, step, 0)

        pltpu.sync_copy(zero_v, h_item.at[pl.ds(sid * sl_item, sl_item)])
        pltpu.sync_copy(zero_v.at[pl.ds(0, sl_small)],
                        h_small.at[pl.ds(sid * sl_small, sl_small)])

        @pl.when(sid == 0)
        def _():
            pltpu.sync_copy(zero_v.at[pl.ds(0, 8)],
                            h_item.at[pl.ds(HALF_ITEM, 8)])
            pltpu.sync_copy(zero_v.at[pl.ds(0, 8)],
                            h_small.at[pl.ds(HALF_SMALL, 8)])

        plsc.subcore_barrier()

        pltpu.sync_copy(item_list.at[pl.ds(base, chunk)], idx_v)
        rebase(HALF_ITEM)
        pltpu.sync_copy(ones_v, h_item.at[loc_v], add=True)
        pltpu.sync_copy(entity_list.at[pl.ds(base, chunk)], idx_v)
        rebase(HALF_SMALL)
        pltpu.sync_copy(ones_v, h_small.at[loc_v], add=True)
        plsc.subcore_barrier()

        pltpu.sync_copy(h_item.at[pl.ds(sid * sl_item, sl_item)],
                        out_item.at[pl.ds(cid * HALF_ITEM + sid * sl_item,
                                          sl_item)])
        pltpu.sync_copy(h_small.at[pl.ds(sid * sl_small, sl_small)],
                        out_entity.at[pl.ds(cid * HALF_SMALL + sid * sl_small,
                                            sl_small)])
        pltpu.sync_copy(zero_v.at[pl.ds(0, sl_small)],
                        h_small.at[pl.ds(sid * sl_small, sl_small)])
        plsc.subcore_barrier()

        pltpu.sync_copy(word_list.at[pl.ds(base, chunk)], idx_v)
        rebase(HALF_SMALL)
        pltpu.sync_copy(ones_v, h_small.at[loc_v], add=True)
        plsc.subcore_barrier()

        pltpu.sync_copy(h_small.at[pl.ds(sid * sl_small, sl_small)],
                        out_word.at[pl.ds(cid * HALF_SMALL + sid * sl_small,
                                          sl_small)])

    return body


def _dot_nt(t, c):
    return lax.dot_general(t, c, (((1,), (1,)), ((), ())),
                           preferred_element_type=jnp.float32)


def _mv_item_kernel(tT_ref, c_ref, out_ref):
    j = pl.program_id(0)

    @pl.when(j == 0)
    def _():
        out_ref[...] = jnp.zeros_like(out_ref)

    out_ref[...] += _dot_nt(tT_ref[...], c_ref[...])


def _mv_small_kernel(tTe_ref, tTw_ref, ce_ref, cw_ref, out_ref):
    j = pl.program_id(0)

    @pl.when(j == 0)
    def _():
        out_ref[...] = jnp.zeros_like(out_ref)

    out_ref[...] += (_dot_nt(tTe_ref[...], ce_ref[...]) +
                     _dot_nt(tTw_ref[...], cw_ref[...]))


def _head_kernel(ei_ref, eew_ref, w_ref, b_ref, out_ref):
    s = (ei_ref[...] + eew_ref[...]) * (1.0 / (3.0 * LIST_LEN))
    out_ref[...] = jnp.sum(s * w_ref[...], axis=0, keepdims=True) + b_ref[...]


def kernel(item_list, entity_list, word_list,
           item_table, entity_table, word_table, W_cls, b_cls):
    info = plsc.get_sparse_core_info()
    nc, ns = info.num_cores, info.num_subcores
    chunk = LIST_LEN // ns

    cnt_item, cnt_entity, cnt_word = _sc_hist_kernel(nc, ns, chunk)(
        item_list.astype(jnp.int32),
        entity_list.astype(jnp.int32),
        word_list.astype(jnp.int32),
    )

    e_item = pl.pallas_call(
        _mv_item_kernel,
        grid=(PAD_ITEM // BLK_ITEM,),
        in_specs=[
            pl.BlockSpec((EMBED_DIM, BLK_ITEM), lambda j: (0, j)),
            pl.BlockSpec((1, BLK_ITEM), lambda j: (0, j)),
        ],
        out_specs=pl.BlockSpec((EMBED_DIM, 1), lambda j: (0, 0)),
        out_shape=jax.ShapeDtypeStruct((EMBED_DIM, 1), jnp.float32),
        compiler_params=pltpu.CompilerParams(
            dimension_semantics=("arbitrary",)),
    )(item_table.T, cnt_item.reshape(1, PAD_ITEM))

    e_ew = pl.pallas_call(
        _mv_small_kernel,
        grid=(PAD_SMALL // BLK_SMALL,),
        in_specs=[
            pl.BlockSpec((EMBED_DIM, BLK_SMALL), lambda j: (0, j)),
            pl.BlockSpec((EMBED_DIM, BLK_SMALL), lambda j: (0, j)),
            pl.BlockSpec((1, BLK_SMALL), lambda j: (0, j)),
            pl.BlockSpec((1, BLK_SMALL), lambda j: (0, j)),
        ],
        out_specs=pl.BlockSpec((EMBED_DIM, 1), lambda j: (0, 0)),
        out_shape=jax.ShapeDtypeStruct((EMBED_DIM, 1), jnp.float32),
        compiler_params=pltpu.CompilerParams(
            dimension_semantics=("arbitrary",)),
    )(entity_table.T, word_table.T,
      cnt_entity.reshape(1, PAD_SMALL), cnt_word.reshape(1, PAD_SMALL))

    out = pl.pallas_call(
        _head_kernel,
        out_shape=jax.ShapeDtypeStruct((1, 2), jnp.float32),
    )(e_item, e_ew, W_cls, b_cls.reshape(1, 2))
    return out


# speedup vs baseline: 4.5425x; 1.0253x over previous
import functools

import jax
import jax.numpy as jnp
from jax import lax
from jax.experimental import pallas as pl
from jax.experimental.pallas import tpu as pltpu
from jax.experimental.pallas import tpu_sc as plsc

LIST_LEN = 16384
EMBED_DIM = 64
N_ITEM = 1000000
N_ENTITY = 100000
N_WORD = 100000
L = 16

BLK_ITEM = 32768
BLK_SMALL = 16384
PAD_ITEM = 31 * BLK_ITEM
PAD_SMALL = 7 * BLK_SMALL
HALF_ITEM = PAD_ITEM // 2
HALF_SMALL = PAD_SMALL // 2


def _sc_hist_kernel(nc, ns, chunk):
    mesh = plsc.VectorSubcoreMesh(core_axis_name="c", subcore_axis_name="s")
    sl_item = HALF_ITEM // ns
    sl_small = HALF_SMALL // ns

    @functools.partial(
        pl.kernel,
        out_type=(
            jax.ShapeDtypeStruct((PAD_ITEM,), jnp.float32),
            jax.ShapeDtypeStruct((PAD_SMALL,), jnp.float32),
            jax.ShapeDtypeStruct((PAD_SMALL,), jnp.float32),
        ),
        mesh=mesh,
        scratch_types=[
            pltpu.VMEM((chunk,), jnp.int32),
            pltpu.VMEM((chunk,), jnp.int32),
            pltpu.VMEM((chunk,), jnp.float32),
            pltpu.VMEM_SHARED((HALF_ITEM + 8,), jnp.float32),
            pltpu.VMEM_SHARED((HALF_SMALL + 8,), jnp.float32),
            pltpu.VMEM_SHARED((HALF_SMALL + 8,), jnp.float32),
        ],
        compiler_params=pltpu.CompilerParams(use_tc_tiling_on_sc=False),
    )
    def body(item_list, entity_list, word_list, zeros_hbm, ones_hbm,
             out_item, out_entity, out_word,
             idx_v, loc_v, ones_v, h_item, h_ent, h_word):
        cid = lax.axis_index("c")
        sid = lax.axis_index("s")
        base = sid * chunk

        def rebase(half):
            lo = cid * half

            def step(i, _):
                v = idx_v[pl.ds(i * L, L)] - lo
                oob = (v < 0) | (v >= half)
                loc_v[pl.ds(i * L, L)] = jnp.where(oob, half, v)
                return 0

            lax.fori_loop(0, chunk // L, step, 0)

        pltpu.sync_copy(zeros_hbm.at[pl.ds(0, sl_item)],
                        h_item.at[pl.ds(sid * sl_item, sl_item)])
        pltpu.sync_copy(zeros_hbm.at[pl.ds(0, sl_small)],
                        h_ent.at[pl.ds(sid * sl_small, sl_small)])
        pltpu.sync_copy(zeros_hbm.at[pl.ds(0, sl_small)],
                        h_word.at[pl.ds(sid * sl_small, sl_small)])
        pltpu.sync_copy(ones_hbm, ones_v)
        pltpu.sync_copy(item_list.at[pl.ds(base, chunk)], idx_v)
        rebase(HALF_ITEM)
        plsc.subcore_barrier()

        pltpu.sync_copy(ones_v, h_item.at[loc_v], add=True)
        pltpu.sync_copy(entity_list.at[pl.ds(base, chunk)], idx_v)
        rebase(HALF_SMALL)
        pltpu.sync_copy(ones_v, h_ent.at[loc_v], add=True)
        pltpu.sync_copy(word_list.at[pl.ds(base, chunk)], idx_v)
        rebase(HALF_SMALL)
        pltpu.sync_copy(ones_v, h_word.at[loc_v], add=True)
        plsc.subcore_barrier()

        pltpu.sync_copy(h_item.at[pl.ds(sid * sl_item, sl_item)],
                        out_item.at[pl.ds(cid * HALF_ITEM + sid * sl_item,
                                          sl_item)])
        pltpu.sync_copy(h_ent.at[pl.ds(sid * sl_small, sl_small)],
                        out_entity.at[pl.ds(cid * HALF_SMALL + sid * sl_small,
                                            sl_small)])
        pltpu.sync_copy(h_word.at[pl.ds(sid * sl_small, sl_small)],
                        out_word.at[pl.ds(cid * HALF_SMALL + sid * sl_small,
                                          sl_small)])

    return body


def _dot_nt(t, c):
    return lax.dot_general(t, c, (((1,), (1,)), ((), ())),
                           preferred_element_type=jnp.float32)


def _mv_item_kernel(tT_ref, c_ref, out_ref):
    j = pl.program_id(0)

    @pl.when(j == 0)
    def _():
        out_ref[...] = jnp.zeros_like(out_ref)

    out_ref[...] += _dot_nt(tT_ref[...], c_ref[...])


def _mv_small_kernel(tTe_ref, tTw_ref, ce_ref, cw_ref, out_ref):
    j = pl.program_id(0)

    @pl.when(j == 0)
    def _():
        out_ref[...] = jnp.zeros_like(out_ref)

    out_ref[...] += (_dot_nt(tTe_ref[...], ce_ref[...]) +
                     _dot_nt(tTw_ref[...], cw_ref[...]))


def _head_kernel(ei_ref, eew_ref, w_ref, b_ref, out_ref):
    s = (ei_ref[...] + eew_ref[...]) * (1.0 / (3.0 * LIST_LEN))
    out_ref[...] = jnp.sum(s * w_ref[...], axis=0, keepdims=True) + b_ref[...]


def kernel(item_list, entity_list, word_list,
           item_table, entity_table, word_table, W_cls, b_cls):
    info = plsc.get_sparse_core_info()
    nc, ns = info.num_cores, info.num_subcores
    chunk = LIST_LEN // ns

    sl_item = HALF_ITEM // ns
    cnt_item, cnt_entity, cnt_word = _sc_hist_kernel(nc, ns, chunk)(
        item_list.astype(jnp.int32),
        entity_list.astype(jnp.int32),
        word_list.astype(jnp.int32),
        jnp.zeros((sl_item,), jnp.float32),
        jnp.ones((chunk,), jnp.float32),
    )

    e_item = pl.pallas_call(
        _mv_item_kernel,
        grid=(PAD_ITEM // BLK_ITEM,),
        in_specs=[
            pl.BlockSpec((EMBED_DIM, BLK_ITEM), lambda j: (0, j)),
            pl.BlockSpec((1, BLK_ITEM), lambda j: (0, j)),
        ],
        out_specs=pl.BlockSpec((EMBED_DIM, 1), lambda j: (0, 0)),
        out_shape=jax.ShapeDtypeStruct((EMBED_DIM, 1), jnp.float32),
        compiler_params=pltpu.CompilerParams(
            dimension_semantics=("arbitrary",)),
    )(item_table.T, cnt_item.reshape(1, PAD_ITEM))

    e_ew = pl.pallas_call(
        _mv_small_kernel,
        grid=(PAD_SMALL // BLK_SMALL,),
        in_specs=[
            pl.BlockSpec((EMBED_DIM, BLK_SMALL), lambda j: (0, j)),
            pl.BlockSpec((EMBED_DIM, BLK_SMALL), lambda j: (0, j)),
            pl.BlockSpec((1, BLK_SMALL), lambda j: (0, j)),
            pl.BlockSpec((1, BLK_SMALL), lambda j: (0, j)),
        ],
        out_specs=pl.BlockSpec((EMBED_DIM, 1), lambda j: (0, 0)),
        out_shape=jax.ShapeDtypeStruct((EMBED_DIM, 1), jnp.float32),
        compiler_params=pltpu.CompilerParams(
            dimension_semantics=("arbitrary",)),
    )(entity_table.T, word_table.T,
      cnt_entity.reshape(1, PAD_SMALL), cnt_word.reshape(1, PAD_SMALL))

    out = pl.pallas_call(
        _head_kernel,
        out_shape=jax.ShapeDtypeStruct((1, 2), jnp.float32),
    )(e_item, e_ew, W_cls, b_cls.reshape(1, 2))
    return out


# speedup vs baseline: 4.6177x; 1.0166x over previous
import functools

import jax
import jax.numpy as jnp
from jax import lax
from jax.experimental import pallas as pl
from jax.experimental.pallas import tpu as pltpu
from jax.experimental.pallas import tpu_sc as plsc

LIST_LEN = 16384
EMBED_DIM = 64
N_ITEM = 1000000
N_ENTITY = 100000
N_WORD = 100000
L = 16

BLK_ITEM = 32768
BLK_SMALL = 16384
PAD_ITEM = 31 * BLK_ITEM
PAD_SMALL = 7 * BLK_SMALL
HALF_ITEM = PAD_ITEM // 2
HALF_SMALL = PAD_SMALL // 2


def _sc_hist_kernel(nc, ns, chunk):
    mesh = plsc.VectorSubcoreMesh(core_axis_name="c", subcore_axis_name="s")
    sl_item = HALF_ITEM // ns
    sl_small = HALF_SMALL // ns

    @functools.partial(
        pl.kernel,
        out_type=(
            jax.ShapeDtypeStruct((PAD_ITEM,), jnp.float32),
            jax.ShapeDtypeStruct((PAD_SMALL,), jnp.float32),
            jax.ShapeDtypeStruct((PAD_SMALL,), jnp.float32),
        ),
        mesh=mesh,
        scratch_types=[
            pltpu.VMEM((chunk,), jnp.int32),
            pltpu.VMEM((chunk,), jnp.int32),
            pltpu.VMEM((chunk,), jnp.int32),
            pltpu.VMEM((chunk,), jnp.int32),
            pltpu.VMEM((chunk,), jnp.int32),
            pltpu.VMEM((chunk,), jnp.int32),
            pltpu.VMEM((chunk,), jnp.float32),
            pltpu.VMEM_SHARED((HALF_ITEM + 8,), jnp.float32),
            pltpu.VMEM_SHARED((HALF_SMALL + 8,), jnp.float32),
            pltpu.VMEM_SHARED((HALF_SMALL + 8,), jnp.float32),
            pltpu.SemaphoreType.DMA,
        ],
        compiler_params=pltpu.CompilerParams(use_tc_tiling_on_sc=False),
    )
    def body(item_list, entity_list, word_list, zeros_hbm, ones_hbm,
             out_item, out_entity, out_word,
             idx_i, idx_e, idx_w, loc_i, loc_e, loc_w, ones_v,
             h_item, h_ent, h_word, sem):
        cid = lax.axis_index("c")
        sid = lax.axis_index("s")
        base = sid * chunk

        def rebase(idx_v, loc_v, half):
            lo = cid * half

            def step(i, _):
                v = idx_v[pl.ds(i * L, L)] - lo
                oob = (v < 0) | (v >= half)
                loc_v[pl.ds(i * L, L)] = jnp.where(oob, half, v)
                return 0

            lax.fori_loop(0, chunk // L, step, 0)

        copies = [
            pltpu.async_copy(zeros_hbm.at[pl.ds(0, sl_item)],
                             h_item.at[pl.ds(sid * sl_item, sl_item)], sem),
            pltpu.async_copy(zeros_hbm.at[pl.ds(0, sl_small)],
                             h_ent.at[pl.ds(sid * sl_small, sl_small)], sem),
            pltpu.async_copy(zeros_hbm.at[pl.ds(0, sl_small)],
                             h_word.at[pl.ds(sid * sl_small, sl_small)], sem),
            pltpu.async_copy(ones_hbm, ones_v, sem),
            pltpu.async_copy(item_list.at[pl.ds(base, chunk)], idx_i, sem),
            pltpu.async_copy(entity_list.at[pl.ds(base, chunk)], idx_e, sem),
            pltpu.async_copy(word_list.at[pl.ds(base, chunk)], idx_w, sem),
        ]
        for c in copies:
            c.wait()
        rebase(idx_i, loc_i, HALF_ITEM)
        rebase(idx_e, loc_e, HALF_SMALL)
        rebase(idx_w, loc_w, HALF_SMALL)
        plsc.subcore_barrier()

        scatters = [
            pltpu.async_copy(ones_v, h_item.at[loc_i], sem, add=True),
            pltpu.async_copy(ones_v, h_ent.at[loc_e], sem, add=True),
            pltpu.async_copy(ones_v, h_word.at[loc_w], sem, add=True),
        ]
        for c in scatters:
            c.wait()
        plsc.subcore_barrier()

        dumps = [
            pltpu.async_copy(h_item.at[pl.ds(sid * sl_item, sl_item)],
                             out_item.at[pl.ds(cid * HALF_ITEM +
                                               sid * sl_item, sl_item)], sem),
            pltpu.async_copy(h_ent.at[pl.ds(sid * sl_small, sl_small)],
                             out_entity.at[pl.ds(cid * HALF_SMALL +
                                                 sid * sl_small, sl_small)],
                             sem),
            pltpu.async_copy(h_word.at[pl.ds(sid * sl_small, sl_small)],
                             out_word.at[pl.ds(cid * HALF_SMALL +
                                               sid * sl_small, sl_small)],
                             sem),
        ]
        for c in dumps:
            c.wait()

    return body


def _dot_nt(t, c):
    return lax.dot_general(t, c, (((1,), (1,)), ((), ())),
                           preferred_element_type=jnp.float32)


def _mv_item_kernel(tT_ref, c_ref, out_ref):
    j = pl.program_id(0)

    @pl.when(j == 0)
    def _():
        out_ref[...] = jnp.zeros_like(out_ref)

    out_ref[...] += _dot_nt(tT_ref[...], c_ref[...])


def _mv_small_kernel(tTe_ref, tTw_ref, ce_ref, cw_ref, out_ref):
    j = pl.program_id(0)

    @pl.when(j == 0)
    def _():
        out_ref[...] = jnp.zeros_like(out_ref)

    out_ref[...] += (_dot_nt(tTe_ref[...], ce_ref[...]) +
                     _dot_nt(tTw_ref[...], cw_ref[...]))


def _head_kernel(ei_ref, eew_ref, w_ref, b_ref, out_ref):
    s = (ei_ref[...] + eew_ref[...]) * (1.0 / (3.0 * LIST_LEN))
    out_ref[...] = jnp.sum(s * w_ref[...], axis=0, keepdims=True) + b_ref[...]


def kernel(item_list, entity_list, word_list,
           item_table, entity_table, word_table, W_cls, b_cls):
    info = plsc.get_sparse_core_info()
    nc, ns = info.num_cores, info.num_subcores
    chunk = LIST_LEN // ns

    sl_item = HALF_ITEM // ns
    cnt_item, cnt_entity, cnt_word = _sc_hist_kernel(nc, ns, chunk)(
        item_list.astype(jnp.int32),
        entity_list.astype(jnp.int32),
        word_list.astype(jnp.int32),
        jnp.zeros((sl_item,), jnp.float32),
        jnp.ones((chunk,), jnp.float32),
    )

    e_item = pl.pallas_call(
        _mv_item_kernel,
        grid=(PAD_ITEM // BLK_ITEM,),
        in_specs=[
            pl.BlockSpec((EMBED_DIM, BLK_ITEM), lambda j: (0, j)),
            pl.BlockSpec((1, BLK_ITEM), lambda j: (0, j)),
        ],
        out_specs=pl.BlockSpec((EMBED_DIM, 1), lambda j: (0, 0)),
        out_shape=jax.ShapeDtypeStruct((EMBED_DIM, 1), jnp.float32),
        compiler_params=pltpu.CompilerParams(
            dimension_semantics=("arbitrary",)),
    )(item_table.T, cnt_item.reshape(1, PAD_ITEM))

    e_ew = pl.pallas_call(
        _mv_small_kernel,
        grid=(PAD_SMALL // BLK_SMALL,),
        in_specs=[
            pl.BlockSpec((EMBED_DIM, BLK_SMALL), lambda j: (0, j)),
            pl.BlockSpec((EMBED_DIM, BLK_SMALL), lambda j: (0, j)),
            pl.BlockSpec((1, BLK_SMALL), lambda j: (0, j)),
            pl.BlockSpec((1, BLK_SMALL), lambda j: (0, j)),
        ],
        out_specs=pl.BlockSpec((EMBED_DIM, 1), lambda j: (0, 0)),
        out_shape=jax.ShapeDtypeStruct((EMBED_DIM, 1), jnp.float32),
        compiler_params=pltpu.CompilerParams(
            dimension_semantics=("arbitrary",)),
    )(entity_table.T, word_table.T,
      cnt_entity.reshape(1, PAD_SMALL), cnt_word.reshape(1, PAD_SMALL))

    out = pl.pallas_call(
        _head_kernel,
        out_shape=jax.ShapeDtypeStruct((1, 2), jnp.float32),
    )(e_item, e_ew, W_cls, b_cls.reshape(1, 2))
    return out


# speedup vs baseline: 5.2767x; 1.1427x over previous
import functools

import jax
import jax.numpy as jnp
from jax import lax
from jax.experimental import pallas as pl
from jax.experimental.pallas import tpu as pltpu
from jax.experimental.pallas import tpu_sc as plsc

LIST_LEN = 16384
EMBED_DIM = 64
N_ITEM = 1000000
N_ENTITY = 100000
N_WORD = 100000
L = 16

BLK_ITEM = 32768
BLK_SMALL = 16384
PAD_ITEM = 31 * BLK_ITEM
PAD_SMALL = 7 * BLK_SMALL
HALF_ITEM = PAD_ITEM // 2
HALF_SMALL = PAD_SMALL // 2
NB_SMALL = PAD_SMALL // BLK_SMALL
SCALE = 1.0 / (3.0 * LIST_LEN)


def _rebase(cid, idx_v, loc_v, half, chunk):
    lo = cid * half

    def step(i, _):
        v = idx_v[pl.ds(i * L, L)] - lo
        oob = (v < 0) | (v >= half)
        loc_v[pl.ds(i * L, L)] = jnp.where(oob, half, v)
        return 0

    lax.fori_loop(0, chunk // L, step, 0)


def _sc_item_kernel(nc, ns, chunk):
    mesh = plsc.VectorSubcoreMesh(core_axis_name="c", subcore_axis_name="s")
    sl = HALF_ITEM // ns

    @functools.partial(
        pl.kernel,
        out_type=jax.ShapeDtypeStruct((PAD_ITEM,), jnp.float32),
        mesh=mesh,
        scratch_types=[
            pltpu.VMEM((chunk,), jnp.int32),
            pltpu.VMEM((chunk,), jnp.int32),
            pltpu.VMEM((chunk,), jnp.float32),
            pltpu.VMEM_SHARED((HALF_ITEM + 8,), jnp.float32),
            pltpu.SemaphoreType.DMA,
        ],
        compiler_params=pltpu.CompilerParams(use_tc_tiling_on_sc=False),
    )
    def body(item_list, zeros_hbm, ones_hbm, out_item,
             idx_v, loc_v, ones_v, h, sem):
        cid = lax.axis_index("c")
        sid = lax.axis_index("s")
        base = sid * chunk
        copies = [
            pltpu.async_copy(zeros_hbm.at[pl.ds(0, sl)],
                             h.at[pl.ds(sid * sl, sl)], sem),
            pltpu.async_copy(ones_hbm, ones_v, sem),
            pltpu.async_copy(item_list.at[pl.ds(base, chunk)], idx_v, sem),
        ]
        for c in copies:
            c.wait()
        _rebase(cid, idx_v, loc_v, HALF_ITEM, chunk)
        plsc.subcore_barrier()
        pltpu.async_copy(ones_v, h.at[loc_v], sem, add=True).wait()
        plsc.subcore_barrier()
        pltpu.async_copy(
            h.at[pl.ds(sid * sl, sl)],
            out_item.at[pl.ds(cid * HALF_ITEM + sid * sl, sl)], sem).wait()

    return body


def _sc_small_kernel(nc, ns, chunk):
    mesh = plsc.VectorSubcoreMesh(core_axis_name="c", subcore_axis_name="s")
    sl = HALF_SMALL // ns

    @functools.partial(
        pl.kernel,
        out_type=(
            jax.ShapeDtypeStruct((PAD_SMALL,), jnp.float32),
            jax.ShapeDtypeStruct((PAD_SMALL,), jnp.float32),
        ),
        mesh=mesh,
        scratch_types=[
            pltpu.VMEM((chunk,), jnp.int32),
            pltpu.VMEM((chunk,), jnp.int32),
            pltpu.VMEM((chunk,), jnp.int32),
            pltpu.VMEM((chunk,), jnp.int32),
            pltpu.VMEM((chunk,), jnp.float32),
            pltpu.VMEM_SHARED((HALF_SMALL + 8,), jnp.float32),
            pltpu.VMEM_SHARED((HALF_SMALL + 8,), jnp.float32),
            pltpu.SemaphoreType.DMA,
        ],
        compiler_params=pltpu.CompilerParams(use_tc_tiling_on_sc=False),
    )
    def body(entity_list, word_list, zeros_hbm, ones_hbm,
             out_entity, out_word,
             idx_e, idx_w, loc_e, loc_w, ones_v, h_ent, h_word, sem):
        cid = lax.axis_index("c")
        sid = lax.axis_index("s")
        base = sid * chunk
        copies = [
            pltpu.async_copy(zeros_hbm.at[pl.ds(0, sl)],
                             h_ent.at[pl.ds(sid * sl, sl)], sem),
            pltpu.async_copy(zeros_hbm.at[pl.ds(0, sl)],
                             h_word.at[pl.ds(sid * sl, sl)], sem),
            pltpu.async_copy(ones_hbm, ones_v, sem),
            pltpu.async_copy(entity_list.at[pl.ds(base, chunk)], idx_e, sem),
            pltpu.async_copy(word_list.at[pl.ds(base, chunk)], idx_w, sem),
        ]
        for c in copies:
            c.wait()
        _rebase(cid, idx_e, loc_e, HALF_SMALL, chunk)
        _rebase(cid, idx_w, loc_w, HALF_SMALL, chunk)
        plsc.subcore_barrier()
        scatters = [
            pltpu.async_copy(ones_v, h_ent.at[loc_e], sem, add=True),
            pltpu.async_copy(ones_v, h_word.at[loc_w], sem, add=True),
        ]
        for c in scatters:
            c.wait()
        plsc.subcore_barrier()
        dumps = [
            pltpu.async_copy(
                h_ent.at[pl.ds(sid * sl, sl)],
                out_entity.at[pl.ds(cid * HALF_SMALL + sid * sl, sl)], sem),
            pltpu.async_copy(
                h_word.at[pl.ds(sid * sl, sl)],
                out_word.at[pl.ds(cid * HALF_SMALL + sid * sl, sl)], sem),
        ]
        for c in dumps:
            c.wait()

    return body


def _dot_nt(t, c):
    return lax.dot_general(t, c, (((1,), (1,)), ((), ())),
                           preferred_element_type=jnp.float32)


def _mv_item_kernel(tT_ref, c_ref, out_ref):
    j = pl.program_id(0)

    @pl.when(j == 0)
    def _():
        out_ref[...] = jnp.zeros_like(out_ref)

    out_ref[...] += _dot_nt(tT_ref[...], c_ref[...])


def _mv_small_head_kernel(tTe_ref, tTw_ref, ce_ref, cw_ref,
                          ei_ref, w_ref, b_ref, out_ref, acc_ref):
    j = pl.program_id(0)

    @pl.when(j == 0)
    def _():
        acc_ref[...] = jnp.zeros_like(acc_ref)

    acc_ref[...] += (_dot_nt(tTe_ref[...], ce_ref[...]) +
                     _dot_nt(tTw_ref[...], cw_ref[...]))

    @pl.when(j == NB_SMALL - 1)
    def _():
        s = (acc_ref[...] + ei_ref[...]) * SCALE
        out_ref[...] = (jnp.sum(s * w_ref[...], axis=0, keepdims=True)
                        + b_ref[...])


def kernel(item_list, entity_list, word_list,
           item_table, entity_table, word_table, W_cls, b_cls):
    info = plsc.get_sparse_core_info()
    nc, ns = info.num_cores, info.num_subcores
    chunk = LIST_LEN // ns
    sl_item = HALF_ITEM // ns
    zeros = jnp.zeros((sl_item,), jnp.float32)
    ones = jnp.ones((chunk,), jnp.float32)

    cnt_item = _sc_item_kernel(nc, ns, chunk)(
        item_list.astype(jnp.int32), zeros, ones)
    cnt_entity, cnt_word = _sc_small_kernel(nc, ns, chunk)(
        entity_list.astype(jnp.int32), word_list.astype(jnp.int32),
        zeros, ones)

    e_item = pl.pallas_call(
        _mv_item_kernel,
        grid=(PAD_ITEM // BLK_ITEM,),
        in_specs=[
            pl.BlockSpec((EMBED_DIM, BLK_ITEM), lambda j: (0, j)),
            pl.BlockSpec((1, BLK_ITEM), lambda j: (0, j)),
        ],
        out_specs=pl.BlockSpec((EMBED_DIM, 1), lambda j: (0, 0)),
        out_shape=jax.ShapeDtypeStruct((EMBED_DIM, 1), jnp.float32),
        compiler_params=pltpu.CompilerParams(
            dimension_semantics=("arbitrary",)),
    )(item_table.T, cnt_item.reshape(1, PAD_ITEM))

    out = pl.pallas_call(
        _mv_small_head_kernel,
        grid=(NB_SMALL,),
        in_specs=[
            pl.BlockSpec((EMBED_DIM, BLK_SMALL), lambda j: (0, j)),
            pl.BlockSpec((EMBED_DIM, BLK_SMALL), lambda j: (0, j)),
            pl.BlockSpec((1, BLK_SMALL), lambda j: (0, j)),
            pl.BlockSpec((1, BLK_SMALL), lambda j: (0, j)),
            pl.BlockSpec((EMBED_DIM, 1), lambda j: (0, 0)),
            pl.BlockSpec((EMBED_DIM, 2), lambda j: (0, 0)),
            pl.BlockSpec((1, 2), lambda j: (0, 0)),
        ],
        out_specs=pl.BlockSpec((1, 2), lambda j: (0, 0)),
        out_shape=jax.ShapeDtypeStruct((1, 2), jnp.float32),
        scratch_shapes=[pltpu.VMEM((EMBED_DIM, 1), jnp.float32)],
        compiler_params=pltpu.CompilerParams(
            dimension_semantics=("arbitrary",)),
    )(entity_table.T, word_table.T,
      cnt_entity.reshape(1, PAD_SMALL), cnt_word.reshape(1, PAD_SMALL),
      e_item, W_cls, b_cls.reshape(1, 2))
    return out


# speedup vs baseline: 5.2836x; 1.0013x over previous
import functools

import jax
import jax.numpy as jnp
from jax import lax
from jax.experimental import pallas as pl
from jax.experimental.pallas import tpu as pltpu
from jax.experimental.pallas import tpu_sc as plsc

LIST_LEN = 16384
EMBED_DIM = 64
N_ITEM = 1000000
N_ENTITY = 100000
N_WORD = 100000
L = 16

BLK_ITEM = 32768
BLK_SMALL = 16384
PAD_ITEM = 31 * BLK_ITEM
PAD_SMALL = 7 * BLK_SMALL
HALF_ITEM = PAD_ITEM // 2
HALF_SMALL = PAD_SMALL // 2
NB_SMALL = PAD_SMALL // BLK_SMALL
SCALE = 1.0 / (3.0 * LIST_LEN)


def _rebase(cid, idx_v, loc_v, half, chunk):
    lo = cid * half

    def step(i, _):
        v = idx_v[pl.ds(i * L, L)] - lo
        oob = (v < 0) | (v >= half)
        loc_v[pl.ds(i * L, L)] = jnp.where(oob, half, v)
        return 0

    lax.fori_loop(0, chunk // L, step, 0)


def _sc_item_kernel(nc, ns, chunk):
    mesh = plsc.VectorSubcoreMesh(core_axis_name="c", subcore_axis_name="s")
    sl = HALF_ITEM // ns

    @functools.partial(
        pl.kernel,
        out_type=jax.ShapeDtypeStruct((PAD_ITEM,), jnp.float32),
        mesh=mesh,
        scratch_types=[
            pltpu.VMEM((chunk,), jnp.int32),
            pltpu.VMEM((chunk,), jnp.int32),
            pltpu.VMEM((chunk,), jnp.float32),
            pltpu.VMEM_SHARED((HALF_ITEM + 8,), jnp.float32),
            pltpu.SemaphoreType.DMA,
        ],
        compiler_params=pltpu.CompilerParams(use_tc_tiling_on_sc=False),
    )
    def body(item_list, zeros_hbm, ones_hbm, out_item,
             idx_v, loc_v, ones_v, h, sem):
        cid = lax.axis_index("c")
        sid = lax.axis_index("s")
        base = sid * chunk
        copies = [
            pltpu.async_copy(zeros_hbm.at[pl.ds(0, sl)],
                             h.at[pl.ds(sid * sl, sl)], sem),
            pltpu.async_copy(ones_hbm, ones_v, sem),
            pltpu.async_copy(item_list.at[pl.ds(base, chunk)], idx_v, sem),
        ]
        for c in copies:
            c.wait()
        _rebase(cid, idx_v, loc_v, HALF_ITEM, chunk)
        plsc.subcore_barrier()
        pltpu.async_copy(ones_v, h.at[loc_v], sem, add=True).wait()
        plsc.subcore_barrier()
        pltpu.async_copy(
            h.at[pl.ds(sid * sl, sl)],
            out_item.at[pl.ds(cid * HALF_ITEM + sid * sl, sl)], sem).wait()

    return body


def _sc_small_kernel(nc, ns, chunk):
    mesh = plsc.VectorSubcoreMesh(core_axis_name="c", subcore_axis_name="s")
    sl = HALF_SMALL // ns

    @functools.partial(
        pl.kernel,
        out_type=(
            jax.ShapeDtypeStruct((PAD_SMALL,), jnp.float32),
            jax.ShapeDtypeStruct((PAD_SMALL,), jnp.float32),
        ),
        mesh=mesh,
        scratch_types=[
            pltpu.VMEM((chunk,), jnp.int32),
            pltpu.VMEM((chunk,), jnp.int32),
            pltpu.VMEM((chunk,), jnp.int32),
            pltpu.VMEM((chunk,), jnp.int32),
            pltpu.VMEM((chunk,), jnp.float32),
            pltpu.VMEM_SHARED((HALF_SMALL + 8,), jnp.float32),
            pltpu.VMEM_SHARED((HALF_SMALL + 8,), jnp.float32),
            pltpu.SemaphoreType.DMA,
        ],
        compiler_params=pltpu.CompilerParams(use_tc_tiling_on_sc=False),
    )
    def body(entity_list, word_list, zeros_hbm, ones_hbm,
             out_entity, out_word,
             idx_e, idx_w, loc_e, loc_w, ones_v, h_ent, h_word, sem):
        cid = lax.axis_index("c")
        sid = lax.axis_index("s")
        base = sid * chunk
        copies = [
            pltpu.async_copy(zeros_hbm.at[pl.ds(0, sl)],
                             h_ent.at[pl.ds(sid * sl, sl)], sem),
            pltpu.async_copy(zeros_hbm.at[pl.ds(0, sl)],
                             h_word.at[pl.ds(sid * sl, sl)], sem),
            pltpu.async_copy(ones_hbm, ones_v, sem),
            pltpu.async_copy(entity_list.at[pl.ds(base, chunk)], idx_e, sem),
            pltpu.async_copy(word_list.at[pl.ds(base, chunk)], idx_w, sem),
        ]
        for c in copies:
            c.wait()
        _rebase(cid, idx_e, loc_e, HALF_SMALL, chunk)
        _rebase(cid, idx_w, loc_w, HALF_SMALL, chunk)
        plsc.subcore_barrier()
        scatters = [
            pltpu.async_copy(ones_v, h_ent.at[loc_e], sem, add=True),
            pltpu.async_copy(ones_v, h_word.at[loc_w], sem, add=True),
        ]
        for c in scatters:
            c.wait()
        plsc.subcore_barrier()
        dumps = [
            pltpu.async_copy(
                h_ent.at[pl.ds(sid * sl, sl)],
                out_entity.at[pl.ds(cid * HALF_SMALL + sid * sl, sl)], sem),
            pltpu.async_copy(
                h_word.at[pl.ds(sid * sl, sl)],
                out_word.at[pl.ds(cid * HALF_SMALL + sid * sl, sl)], sem),
        ]
        for c in dumps:
            c.wait()

    return body


def _dot_nt(t, c):
    return lax.dot_general(t, c, (((1,), (1,)), ((), ())),
                           preferred_element_type=jnp.float32)


def _mv_item_kernel(t0_ref, t1_ref, c_ref, out_ref):
    j = pl.program_id(0)

    @pl.when(j == 0)
    def _():
        out_ref[...] = jnp.zeros_like(out_ref)

    c = c_ref[...]
    out_ref[0:32, :] += _dot_nt(t0_ref[...], c)
    out_ref[32:64, :] += _dot_nt(t1_ref[...], c)


def _mv_small_head_kernel(tTe_ref, tTw_ref, ce_ref, cw_ref,
                          ei_ref, w_ref, b_ref, out_ref, acc_ref):
    j = pl.program_id(0)

    @pl.when(j == 0)
    def _():
        acc_ref[...] = jnp.zeros_like(acc_ref)

    acc_ref[...] += (_dot_nt(tTe_ref[...], ce_ref[...]) +
                     _dot_nt(tTw_ref[...], cw_ref[...]))

    @pl.when(j == NB_SMALL - 1)
    def _():
        s = (acc_ref[...] + ei_ref[...]) * SCALE
        out_ref[...] = (jnp.sum(s * w_ref[...], axis=0, keepdims=True)
                        + b_ref[...])


def kernel(item_list, entity_list, word_list,
           item_table, entity_table, word_table, W_cls, b_cls):
    info = plsc.get_sparse_core_info()
    nc, ns = info.num_cores, info.num_subcores
    chunk = LIST_LEN // ns
    sl_item = HALF_ITEM // ns
    zeros = jnp.zeros((sl_item,), jnp.float32)
    ones = jnp.ones((chunk,), jnp.float32)

    cnt_item = _sc_item_kernel(nc, ns, chunk)(
        item_list.astype(jnp.int32), zeros, ones)
    cnt_entity, cnt_word = _sc_small_kernel(nc, ns, chunk)(
        entity_list.astype(jnp.int32), word_list.astype(jnp.int32),
        zeros, ones)

    e_item = pl.pallas_call(
        _mv_item_kernel,
        grid=(PAD_ITEM // BLK_ITEM,),
        in_specs=[
            pl.BlockSpec((EMBED_DIM // 2, BLK_ITEM), lambda j: (0, j)),
            pl.BlockSpec((EMBED_DIM // 2, BLK_ITEM), lambda j: (1, j)),
            pl.BlockSpec((1, BLK_ITEM), lambda j: (0, j)),
        ],
        out_specs=pl.BlockSpec((EMBED_DIM, 1), lambda j: (0, 0)),
        out_shape=jax.ShapeDtypeStruct((EMBED_DIM, 1), jnp.float32),
        compiler_params=pltpu.CompilerParams(
            dimension_semantics=("arbitrary",)),
    )(item_table.T, item_table.T, cnt_item.reshape(1, PAD_ITEM))

    out = pl.pallas_call(
        _mv_small_head_kernel,
        grid=(NB_SMALL,),
        in_specs=[
            pl.BlockSpec((EMBED_DIM, BLK_SMALL), lambda j: (0, j)),
            pl.BlockSpec((EMBED_DIM, BLK_SMALL), lambda j: (0, j)),
            pl.BlockSpec((1, BLK_SMALL), lambda j: (0, j)),
            pl.BlockSpec((1, BLK_SMALL), lambda j: (0, j)),
            pl.BlockSpec((EMBED_DIM, 1), lambda j: (0, 0)),
            pl.BlockSpec((EMBED_DIM, 2), lambda j: (0, 0)),
            pl.BlockSpec((1, 2), lambda j: (0, 0)),
        ],
        out_specs=pl.BlockSpec((1, 2), lambda j: (0, 0)),
        out_shape=jax.ShapeDtypeStruct((1, 2), jnp.float32),
        scratch_shapes=[pltpu.VMEM((EMBED_DIM, 1), jnp.float32)],
        compiler_params=pltpu.CompilerParams(
            dimension_semantics=("arbitrary",)),
    )(entity_table.T, word_table.T,
      cnt_entity.reshape(1, PAD_SMALL), cnt_word.reshape(1, PAD_SMALL),
      e_item, W_cls, b_cls.reshape(1, 2))
    return out
